# baseline (speedup 1.0000x reference)
import functools

import jax
import jax.numpy as jnp
from jax.experimental import pallas as pl
from jax.experimental.pallas import tpu as pltpu

f32 = jnp.float32


def _conv_body(*refs, TH, W, Wp, Cin, Cout, H, act, n_tiles, pool, has_tb,
               hcw=False):
    if has_tb:
        xm, xtop, xbot, w_ref, sc_ref, sh_ref, tb_ref, o_ref, xbuf = refs
    else:
        xm, xtop, xbot, w_ref, sc_ref, sh_ref, o_ref, xbuf = refs
        tb_ref = None
    i = pl.program_id(1)
    nf = (i > 0).astype(f32)
    nl = (i < n_tiles - 1).astype(f32)
    dt = xbuf.dtype
    xbuf[:, W:Wp, :] = jnp.zeros((TH + 4, Wp - W, Cin), dt)
    xbuf[0:1, 0:W, :] = jnp.zeros((1, W, Cin), dt)
    xbuf[TH + 3:TH + 4, 0:W, :] = jnp.zeros((1, W, Cin), dt)
    xbuf[2:TH + 2, 0:W, :] = xm[0]
    xbuf[1:2, 0:W, :] = (xtop[0] * nf).astype(dt)
    xbuf[TH + 2:TH + 3, 0:W, :] = (xbot[0] * nl).astype(dt)

    flat = xbuf[...].reshape((TH + 4) * Wp, Cin)
    acc = jnp.zeros((TH * Wp, Cout), f32)
    for dw in (-1, 0, 1):
        xs = flat[Wp + dw: Wp + dw + (TH + 2) * Wp, :]
        for kh in range(3):
            acc = acc + jnp.dot(xs[kh * Wp:(kh + TH) * Wp, :],
                                w_ref[kh * 3 + dw + 1],
                                preferred_element_type=f32)
    acc = acc * sc_ref[...] + sh_ref[...]
    a3 = acc.reshape(TH, Wp, Cout)

    if tb_ref is not None:
        tb = tb_ref[0]

        def v(k):
            return tb[k:k + 1, :].reshape(1, 1, Cout)

        rr = jax.lax.broadcasted_iota(jnp.int32, (TH, 1, 1), 0) + i * TH
        mt = (rr == 0).astype(f32)
        mb = (rr == H - 1).astype(f32)
        cc = jax.lax.broadcasted_iota(jnp.int32, (1, Wp, 1), 1)
        ml = (cc == 0).astype(f32)
        mr = (cc == W - 1).astype(f32)
        a3 = a3 + (v(0) + mt * v(1) + mb * v(2))
        a3 = a3 + ml * (v(3) + mt * v(5) + mb * v(7))
        a3 = a3 + mr * (v(4) + mt * v(6) + mb * v(8))

    if act == "relu":
        a3 = jnp.maximum(a3, 0.0)
    elif act == "sigmoid":
        a3 = 1.0 / (1.0 + jnp.exp(-a3))

    if pool:
        a = a3[:, 0:W, :].reshape(TH // 2, 2, W, Cout).max(axis=1)
        a = a.reshape(TH // 2, W // 2, 2, Cout).max(axis=2)
        o_ref[0] = a.astype(o_ref.dtype)
    elif hcw:
        o_ref[0] = jnp.swapaxes(a3[:, 0:W, :], 1, 2).astype(o_ref.dtype)
    else:
        o_ref[0] = a3[:, 0:W, :].astype(o_ref.dtype)


def _conv(x, w9, scale, shift, *, act, tb=None, pool=False, hcw=False, TH=32):
    B, H, W, Cin = x.shape
    Cout = w9.shape[-1]
    TH = min(TH, H)
    n_tiles = H // TH
    Wp = W + 8
    body = functools.partial(
        _conv_body, TH=TH, W=W, Wp=Wp, Cin=Cin, Cout=Cout, H=H, act=act,
        n_tiles=n_tiles, pool=pool, has_tb=tb is not None, hcw=hcw)
    in_specs = [
        pl.BlockSpec((1, TH, W, Cin), lambda b, i: (b, i, 0, 0)),
        pl.BlockSpec((1, 1, W, Cin),
                     lambda b, i: (b, jnp.maximum(i * TH - 1, 0), 0, 0)),
        pl.BlockSpec((1, 1, W, Cin),
                     lambda b, i: (b, jnp.minimum(i * TH + TH, H - 1), 0, 0)),
        pl.BlockSpec((9, Cin, Cout), lambda b, i: (0, 0, 0)),
        pl.BlockSpec((1, Cout), lambda b, i: (0, 0)),
        pl.BlockSpec((1, Cout), lambda b, i: (0, 0)),
    ]
    inputs = [x, x, x, w9, scale.reshape(1, Cout), shift.reshape(1, Cout)]
    if tb is not None:
        in_specs.append(pl.BlockSpec((1, 9, Cout), lambda b, i: (b, 0, 0)))
        inputs.append(tb)
    out_dt = x.dtype
    if pool:
        out_shape = jax.ShapeDtypeStruct((B, H // 2, W // 2, Cout), out_dt)
        out_spec = pl.BlockSpec((1, TH // 2, W // 2, Cout),
                                lambda b, i: (b, i, 0, 0))
    elif hcw:
        out_shape = jax.ShapeDtypeStruct((B, H, Cout, W), out_dt)
        out_spec = pl.BlockSpec((1, TH, Cout, W), lambda b, i: (b, i, 0, 0))
    else:
        out_shape = jax.ShapeDtypeStruct((B, H, W, Cout), out_dt)
        out_spec = pl.BlockSpec((1, TH, W, Cout), lambda b, i: (b, i, 0, 0))
    return pl.pallas_call(
        body,
        out_shape=out_shape,
        grid_spec=pltpu.PrefetchScalarGridSpec(
            num_scalar_prefetch=0,
            grid=(B, n_tiles),
            in_specs=in_specs,
            out_specs=out_spec,
            scratch_shapes=[pltpu.VMEM((TH + 4, Wp, Cin), x.dtype)]),
        compiler_params=pltpu.CompilerParams(
            dimension_semantics=("parallel", "parallel")),
    )(*inputs)


def _pair_body(xm, xt, xb, w1_ref, sc1_ref, sh1_ref, w2_ref, sc2_ref,
               sh2_ref, tb_ref, o_ref, xbuf1, xbuf2, *, TH, W, Wp, C0, C1,
               C2, H, n_tiles):
    i = pl.program_id(1)
    dt = xbuf1.dtype
    nf = (i > 0).astype(f32)
    nl = (i < n_tiles - 1).astype(f32)
    xbuf1[:, W:Wp, :] = jnp.zeros((TH + 6, Wp - W, C0), dt)
    xbuf1[0:1, 0:W, :] = jnp.zeros((1, W, C0), dt)
    xbuf1[TH + 5:TH + 6, 0:W, :] = jnp.zeros((1, W, C0), dt)
    xbuf1[3:TH + 3, 0:W, :] = xm[0]
    nf2 = (i * TH - 2 >= 0).astype(f32)
    xbuf1[1:3, 0:W, :] = (xt[0] * nf2).astype(dt)
    nl2 = (i * TH + TH + 1 <= H - 1).astype(f32)
    xbuf1[TH + 3:TH + 5, 0:W, :] = (xb[0] * nl2).astype(dt)

    flat1 = xbuf1[...].reshape((TH + 6) * Wp, C0)
    acc1 = jnp.zeros(((TH + 2) * Wp, C1), f32)
    for dw in (-1, 0, 1):
        xs = flat1[Wp + dw: Wp + dw + (TH + 4) * Wp, :]
        for kh in range(3):
            acc1 = acc1 + jnp.dot(xs[kh * Wp:(kh + TH + 2) * Wp, :],
                                  w1_ref[kh * 3 + dw + 1],
                                  preferred_element_type=f32)
    a1 = jnp.maximum(acc1 * sc1_ref[...] + sh1_ref[...], 0.0)
    a1 = a1.reshape(TH + 2, Wp, C1)

    xbuf2[:, W:Wp, :] = jnp.zeros((TH + 4, Wp - W, C1), dt)
    xbuf2[0:1, 0:W, :] = jnp.zeros((1, W, C1), dt)
    xbuf2[TH + 3:TH + 4, 0:W, :] = jnp.zeros((1, W, C1), dt)
    xbuf2[2:TH + 2, 0:W, :] = a1[1:TH + 1, 0:W, :].astype(dt)
    xbuf2[1:2, 0:W, :] = (a1[0:1, 0:W, :] * nf).astype(dt)
    xbuf2[TH + 2:TH + 3, 0:W, :] = (a1[TH + 1:TH + 2, 0:W, :] * nl).astype(dt)

    flat2 = xbuf2[...].reshape((TH + 4) * Wp, C1)
    acc2 = jnp.zeros((TH * Wp, C2), f32)
    for dw in (-1, 0, 1):
        xs = flat2[Wp + dw: Wp + dw + (TH + 2) * Wp, :]
        for kh in range(3):
            acc2 = acc2 + jnp.dot(xs[kh * Wp:(kh + TH) * Wp, :],
                                  w2_ref[kh * 3 + dw + 1],
                                  preferred_element_type=f32)
    acc2 = acc2 * sc2_ref[...] + sh2_ref[...]
    a3 = acc2.reshape(TH, Wp, C2)

    tb = tb_ref[0]

    def v(k):
        return tb[k:k + 1, :].reshape(1, 1, C2)

    rr = jax.lax.broadcasted_iota(jnp.int32, (TH, 1, 1), 0) + i * TH
    mt = (rr == 0).astype(f32)
    mb = (rr == H - 1).astype(f32)
    cc = jax.lax.broadcasted_iota(jnp.int32, (1, Wp, 1), 1)
    ml = (cc == 0).astype(f32)
    mr = (cc == W - 1).astype(f32)
    a3 = a3 + (v(0) + mt * v(1) + mb * v(2))
    a3 = a3 + ml * (v(3) + mt * v(5) + mb * v(7))
    a3 = a3 + mr * (v(4) + mt * v(6) + mb * v(8))
    a3 = jnp.maximum(a3, 0.0)
    o_ref[0] = a3[:, 0:W, :].astype(o_ref.dtype)


def _conv_pair(x, w1_9, sc1, sh1, w2_9, sc2, sh2, tb, *, TH=32):
    B, H, W, C0 = x.shape
    C1 = w1_9.shape[-1]
    C2 = w2_9.shape[-1]
    TH = min(TH, H)
    n_tiles = H // TH
    Wp = W + 8
    body = functools.partial(_pair_body, TH=TH, W=W, Wp=Wp, C0=C0, C1=C1,
                             C2=C2, H=H, n_tiles=n_tiles)
    in_specs = [
        pl.BlockSpec((1, TH, W, C0), lambda b, i: (b, i, 0, 0)),
        pl.BlockSpec((1, 2, W, C0),
                     lambda b, i: (b, jnp.maximum(i * (TH // 2) - 1, 0),
                                   0, 0)),
        pl.BlockSpec((1, 2, W, C0),
                     lambda b, i: (b, jnp.minimum(i * (TH // 2) + TH // 2,
                                                  H // 2 - 1), 0, 0)),
        pl.BlockSpec((9, C0, C1), lambda b, i: (0, 0, 0)),
        pl.BlockSpec((1, C1), lambda b, i: (0, 0)),
        pl.BlockSpec((1, C1), lambda b, i: (0, 0)),
        pl.BlockSpec((9, C1, C2), lambda b, i: (0, 0, 0)),
        pl.BlockSpec((1, C2), lambda b, i: (0, 0)),
        pl.BlockSpec((1, C2), lambda b, i: (0, 0)),
        pl.BlockSpec((1, 9, C2), lambda b, i: (b, 0, 0)),
    ]
    inputs = [x, x, x, w1_9, sc1.reshape(1, C1), sh1.reshape(1, C1),
              w2_9, sc2.reshape(1, C2), sh2.reshape(1, C2), tb]
    return pl.pallas_call(
        body,
        out_shape=jax.ShapeDtypeStruct((B, H, W, C2), x.dtype),
        grid_spec=pltpu.PrefetchScalarGridSpec(
            num_scalar_prefetch=0,
            grid=(B, n_tiles),
            in_specs=in_specs,
            out_specs=pl.BlockSpec((1, TH, W, C2), lambda b, i: (b, i, 0, 0)),
            scratch_shapes=[pltpu.VMEM((TH + 6, Wp, C0), x.dtype),
                            pltpu.VMEM((TH + 4, Wp, C1), x.dtype)]),
        compiler_params=pltpu.CompilerParams(
            dimension_semantics=("parallel", "parallel")),
    )(*inputs)


def _lin_body(x_ref, w_ref, b_ref, o_ref):
    o_ref[...] = (jnp.dot(x_ref[...], w_ref[...],
                          preferred_element_type=f32) + b_ref[...])


def _linear(x, w, b):
    B, K = x.shape
    N = w.shape[-1]
    return pl.pallas_call(
        _lin_body,
        out_shape=jax.ShapeDtypeStruct((B, N), f32),
        grid=(1,),
        in_specs=[pl.BlockSpec((B, K), lambda i: (0, 0)),
                  pl.BlockSpec((K, N), lambda i: (0, 0)),
                  pl.BlockSpec((1, N), lambda i: (0, 0))],
        out_specs=pl.BlockSpec((B, N), lambda i: (0, 0)),
    )(x, w, b.reshape(1, N))


def _wstar9(wt):
    full = wt.sum((0, 1))
    top = -wt[0].sum(0)
    bot = -wt[2].sum(0)
    left = -wt[:, 0].sum(0)
    right = -wt[:, 2].sum(0)
    return jnp.stack([full, top, bot, left, right,
                      wt[0, 0], wt[0, 2], wt[2, 0], wt[2, 2]])


def _bilin_mat(n_in, n_out):
    i = jnp.arange(n_out, dtype=f32)
    src = i * (n_in - 1) / (n_out - 1)
    i0 = jnp.clip(jnp.floor(src).astype(jnp.int32), 0, n_in - 2)
    frac = src - i0.astype(f32)
    rows = jnp.arange(n_out)
    M = jnp.zeros((n_out, n_in), f32)
    M = M.at[rows, i0].add(1.0 - frac)
    M = M.at[rows, i0 + 1].add(frac)
    return M


def _upconv_body(x_ref, mh_ref, mwt_ref, w9t_ref, sc_ref, sh_ref, o_ref,
                 zbuf, *, TH, Hin, Win, C):
    i = pl.program_id(1)
    xflat = x_ref[0].reshape(Hin, C * Win)
    mh = mh_ref[pl.ds(i * TH, TH + 2), :]
    uph = jnp.dot(mh, xflat, preferred_element_type=f32)
    uph3 = uph.reshape(TH + 2, C, Win)
    w9t = w9t_ref[...]
    for r in range(TH + 2):
        zbuf[r] = jnp.dot(w9t, uph3[r], preferred_element_type=f32)
    zb = zbuf[...]
    acc = jnp.zeros((TH, 2 * Win), f32)
    for kh in range(3):
        for kw in range(3):
            zs = zb[kh:kh + TH, 3 * kh + kw, :]
            acc = acc + jnp.dot(zs, mwt_ref[kw],
                                preferred_element_type=f32)
    acc = acc * sc_ref[0, 0] + sh_ref[0, 0]
    o_ref[0, 0] = 1.0 / (1.0 + jnp.exp(-acc))


def _upconv(x_hcw, w, scale, shift, *, TH=32):
    B, Hin, C, Win = x_hcw.shape
    Hout, Wout = 2 * Hin, 2 * Win
    TH = min(TH, Hout)
    n_tiles = Hout // TH
    Mh = _bilin_mat(Hin, Hout)
    mh_pad = jnp.zeros((Hout + 8, Hin), f32).at[1:Hout + 1, :].set(Mh)
    Mw = _bilin_mat(Win, Wout)
    mw_pad = jnp.zeros((Wout + 2, Win), f32).at[1:Wout + 1, :].set(Mw)
    mwt = jnp.stack([mw_pad[kw:kw + Wout, :].T for kw in range(3)])
    w9t = w.reshape(9, C)
    body = functools.partial(_upconv_body, TH=TH, Hin=Hin, Win=Win, C=C)
    return pl.pallas_call(
        body,
        out_shape=jax.ShapeDtypeStruct((B, 1, Hout, Wout), f32),
        grid_spec=pltpu.PrefetchScalarGridSpec(
            num_scalar_prefetch=0,
            grid=(B, n_tiles),
            in_specs=[
                pl.BlockSpec((1, Hin, C, Win), lambda b, i: (b, 0, 0, 0)),
                pl.BlockSpec((Hout + 8, Hin), lambda b, i: (0, 0)),
                pl.BlockSpec((3, Win, Wout), lambda b, i: (0, 0, 0)),
                pl.BlockSpec((9, C), lambda b, i: (0, 0)),
                pl.BlockSpec((1, 1), lambda b, i: (0, 0)),
                pl.BlockSpec((1, 1), lambda b, i: (0, 0)),
            ],
            out_specs=pl.BlockSpec((1, 1, TH, Wout),
                                   lambda b, i: (b, 0, i, 0)),
            scratch_shapes=[pltpu.VMEM((TH + 2, 9, Win), f32)]),
        compiler_params=pltpu.CompilerParams(
            dimension_semantics=("parallel", "parallel")),
    )(x_hcw, mh_pad, mwt, w9t, scale.reshape(1, 1), shift.reshape(1, 1))


def kernel(x_nchw, text, t_w, t_b, e1_w, e1_scale, e1_shift, e2_wx, e2_wt,
           e2_scale, e2_shift, e3_wx, e3_wt, e3_scale, e3_shift, d1_w,
           d1_scale, d1_shift, d2_wx, d2_wt, d2_scale, d2_shift, d3a_wx,
           d3a_wt, d3a_scale, d3a_shift, d3b_w, d3b_scale, d3b_shift):
    B = x_nchw.shape[0]
    x = jnp.transpose(x_nchw, (0, 2, 3, 1)).astype(jnp.bfloat16)
    x = jnp.pad(x, ((0, 0), (0, 0), (0, 0), (0, 5)))
    text = text.astype(f32)

    layers = [(e2_wt, e2_scale, 0), (e3_wt, e3_scale, 64),
              (d2_wt, d2_scale, 128), (d3a_wt, d3a_scale, 192)]
    segs, bsegs = [], []
    for wt_, sc_, off in layers:
        ws = _wstar9(wt_) * sc_
        segs.append(jnp.einsum('kc,tco->kto', t_w[:, off:off + 64],
                               ws).reshape(512, -1))
        bsegs.append(jnp.einsum('c,tco->to', t_b[off:off + 64],
                                ws).reshape(-1))
    tball = _linear(text, jnp.concatenate(segs, axis=1),
                    jnp.concatenate(bsegs))
    tbs, o = [], 0
    for wt_, _, _ in layers:
        n = 9 * wt_.shape[-1]
        tbs.append(tball[:, o:o + n].reshape(B, 9, wt_.shape[-1]))
        o += n
    tb_e2, tb_e3, tb_d2, tb_d3a = tbs

    def r9(w):
        return w.reshape(9, w.shape[2], w.shape[3])

    e1w = jnp.pad(e1_w, ((0, 0), (0, 0), (0, 5), (0, 0)))
    h = _conv_pair(x, r9(e1w), e1_scale, e1_shift,
                   r9(e2_wx), e2_scale, e2_shift, tb_e2)
    h = _conv(h, r9(e3_wx), e3_scale, e3_shift, act='relu', tb=tb_e3,
              pool=True)
    h = _conv_pair(h, r9(d1_w), d1_scale, d1_shift,
                   r9(d2_wx), d2_scale, d2_shift, tb_d2)
    h = _conv(h, r9(d3a_wx), d3a_scale, d3a_shift, act='relu', tb=tb_d3a,
              hcw=True)
    return _upconv(h, d3b_w, d3b_scale, d3b_shift)


# speedup vs baseline: 1.5515x; 1.5515x over previous
import functools

import jax
import jax.numpy as jnp
from jax.experimental import pallas as pl
from jax.experimental.pallas import tpu as pltpu

f32 = jnp.float32


def _conv_body(*refs, TH, W, Wp, Cin, Cout, H, act, n_tiles, pool, has_tb,
               hcw=False):
    if has_tb:
        xm, xtop, xbot, w_ref, sc_ref, sh_ref, tb_ref, o_ref, xbuf = refs
    else:
        xm, xtop, xbot, w_ref, sc_ref, sh_ref, o_ref, xbuf = refs
        tb_ref = None
    i = pl.program_id(1)
    nf = (i > 0).astype(f32)
    nl = (i < n_tiles - 1).astype(f32)
    dt = xbuf.dtype
    xbuf[:, W:Wp, :] = jnp.zeros((TH + 4, Wp - W, Cin), dt)
    xbuf[0:1, 0:W, :] = jnp.zeros((1, W, Cin), dt)
    xbuf[TH + 3:TH + 4, 0:W, :] = jnp.zeros((1, W, Cin), dt)
    xbuf[2:TH + 2, 0:W, :] = xm[0]
    xbuf[1:2, 0:W, :] = (xtop[0] * nf).astype(dt)
    xbuf[TH + 2:TH + 3, 0:W, :] = (xbot[0] * nl).astype(dt)

    flat = xbuf[...].reshape((TH + 4) * Wp, Cin)
    acc = jnp.zeros((TH * Wp, Cout), f32)
    for dw in (-1, 0, 1):
        xs = flat[Wp + dw: Wp + dw + (TH + 2) * Wp, :]
        for kh in range(3):
            acc = acc + jnp.dot(xs[kh * Wp:(kh + TH) * Wp, :],
                                w_ref[kh * 3 + dw + 1],
                                preferred_element_type=f32)
    acc = acc * sc_ref[...] + sh_ref[...]
    a3 = acc.reshape(TH, Wp, Cout)

    if tb_ref is not None:
        tb = tb_ref[0]

        def v(k):
            return tb[k:k + 1, :].reshape(1, 1, Cout)

        rr = jax.lax.broadcasted_iota(jnp.int32, (TH, 1, 1), 0) + i * TH
        mt = (rr == 0).astype(f32)
        mb = (rr == H - 1).astype(f32)
        cc = jax.lax.broadcasted_iota(jnp.int32, (1, Wp, 1), 1)
        ml = (cc == 0).astype(f32)
        mr = (cc == W - 1).astype(f32)
        a3 = a3 + (v(0) + mt * v(1) + mb * v(2))
        a3 = a3 + ml * (v(3) + mt * v(5) + mb * v(7))
        a3 = a3 + mr * (v(4) + mt * v(6) + mb * v(8))

    if act == "relu":
        a3 = jnp.maximum(a3, 0.0)
    elif act == "sigmoid":
        a3 = 1.0 / (1.0 + jnp.exp(-a3))

    if pool:
        a = a3[:, 0:W, :].reshape(TH // 2, 2, W, Cout).max(axis=1)
        a = a.reshape(TH // 2, W // 2, 2, Cout).max(axis=2)
        o_ref[0] = a.astype(o_ref.dtype)
    elif hcw:
        o_ref[0] = jnp.swapaxes(a3[:, 0:W, :], 1, 2).astype(o_ref.dtype)
    else:
        o_ref[0] = a3[:, 0:W, :].astype(o_ref.dtype)


def _conv(x, w9, scale, shift, *, act, tb=None, pool=False, hcw=False, TH=32):
    B, H, W, Cin = x.shape
    Cout = w9.shape[-1]
    TH = min(TH, H)
    n_tiles = H // TH
    Wp = W + 8
    body = functools.partial(
        _conv_body, TH=TH, W=W, Wp=Wp, Cin=Cin, Cout=Cout, H=H, act=act,
        n_tiles=n_tiles, pool=pool, has_tb=tb is not None, hcw=hcw)
    in_specs = [
        pl.BlockSpec((1, TH, W, Cin), lambda b, i: (b, i, 0, 0)),
        pl.BlockSpec((1, 1, W, Cin),
                     lambda b, i: (b, jnp.maximum(i * TH - 1, 0), 0, 0)),
        pl.BlockSpec((1, 1, W, Cin),
                     lambda b, i: (b, jnp.minimum(i * TH + TH, H - 1), 0, 0)),
        pl.BlockSpec((9, Cin, Cout), lambda b, i: (0, 0, 0)),
        pl.BlockSpec((1, Cout), lambda b, i: (0, 0)),
        pl.BlockSpec((1, Cout), lambda b, i: (0, 0)),
    ]
    inputs = [x, x, x, w9, scale.reshape(1, Cout), shift.reshape(1, Cout)]
    if tb is not None:
        in_specs.append(pl.BlockSpec((1, 9, Cout), lambda b, i: (b, 0, 0)))
        inputs.append(tb)
    out_dt = x.dtype
    if pool:
        out_shape = jax.ShapeDtypeStruct((B, H // 2, W // 2, Cout), out_dt)
        out_spec = pl.BlockSpec((1, TH // 2, W // 2, Cout),
                                lambda b, i: (b, i, 0, 0))
    elif hcw:
        out_shape = jax.ShapeDtypeStruct((B, H, Cout, W), out_dt)
        out_spec = pl.BlockSpec((1, TH, Cout, W), lambda b, i: (b, i, 0, 0))
    else:
        out_shape = jax.ShapeDtypeStruct((B, H, W, Cout), out_dt)
        out_spec = pl.BlockSpec((1, TH, W, Cout), lambda b, i: (b, i, 0, 0))
    return pl.pallas_call(
        body,
        out_shape=out_shape,
        grid_spec=pltpu.PrefetchScalarGridSpec(
            num_scalar_prefetch=0,
            grid=(B, n_tiles),
            in_specs=in_specs,
            out_specs=out_spec,
            scratch_shapes=[pltpu.VMEM((TH + 4, Wp, Cin), x.dtype)]),
        compiler_params=pltpu.CompilerParams(
            dimension_semantics=("parallel", "parallel")),
    )(*inputs)


def _lin_body(x_ref, w_ref, b_ref, o_ref):
    o_ref[...] = (jnp.dot(x_ref[...], w_ref[...],
                          preferred_element_type=f32) + b_ref[...])


def _linear(x, w, b):
    B, K = x.shape
    N = w.shape[-1]
    return pl.pallas_call(
        _lin_body,
        out_shape=jax.ShapeDtypeStruct((B, N), f32),
        grid=(1,),
        in_specs=[pl.BlockSpec((B, K), lambda i: (0, 0)),
                  pl.BlockSpec((K, N), lambda i: (0, 0)),
                  pl.BlockSpec((1, N), lambda i: (0, 0))],
        out_specs=pl.BlockSpec((B, N), lambda i: (0, 0)),
    )(x, w, b.reshape(1, N))


def _wstar9(wt):
    full = wt.sum((0, 1))
    top = -wt[0].sum(0)
    bot = -wt[2].sum(0)
    left = -wt[:, 0].sum(0)
    right = -wt[:, 2].sum(0)
    return jnp.stack([full, top, bot, left, right,
                      wt[0, 0], wt[0, 2], wt[2, 0], wt[2, 2]])


def _bilin_mat(n_in, n_out):
    i = jnp.arange(n_out, dtype=f32)
    src = i * (n_in - 1) / (n_out - 1)
    i0 = jnp.clip(jnp.floor(src).astype(jnp.int32), 0, n_in - 2)
    frac = src - i0.astype(f32)
    rows = jnp.arange(n_out)
    M = jnp.zeros((n_out, n_in), f32)
    M = M.at[rows, i0].add(1.0 - frac)
    M = M.at[rows, i0 + 1].add(frac)
    return M


def _upconv_body(x_ref, mh_ref, mwt_ref, w9t_ref, sc_ref, sh_ref, o_ref,
                 zbuf, *, TH, Hin, Win, C):
    i = pl.program_id(1)
    xflat = x_ref[0].reshape(Hin, C * Win)
    mh = mh_ref[pl.ds(i * TH, TH + 2), :]
    uph = jnp.dot(mh, xflat, preferred_element_type=f32)
    uph3 = uph.reshape(TH + 2, C, Win)
    w9t = w9t_ref[...]
    for r in range(TH + 2):
        zbuf[r] = jnp.dot(w9t, uph3[r], preferred_element_type=f32)
    zb = zbuf[...]
    acc = jnp.zeros((TH, 2 * Win), f32)
    for kh in range(3):
        for kw in range(3):
            zs = zb[kh:kh + TH, 3 * kh + kw, :]
            acc = acc + jnp.dot(zs, mwt_ref[kw],
                                preferred_element_type=f32)
    acc = acc * sc_ref[0, 0] + sh_ref[0, 0]
    o_ref[0, 0] = 1.0 / (1.0 + jnp.exp(-acc))


def _upconv(x_hcw, w, scale, shift, *, TH=32):
    B, Hin, C, Win = x_hcw.shape
    Hout, Wout = 2 * Hin, 2 * Win
    TH = min(TH, Hout)
    n_tiles = Hout // TH
    Mh = _bilin_mat(Hin, Hout)
    mh_pad = jnp.zeros((Hout + 8, Hin), f32).at[1:Hout + 1, :].set(Mh)
    Mw = _bilin_mat(Win, Wout)
    mw_pad = jnp.zeros((Wout + 2, Win), f32).at[1:Wout + 1, :].set(Mw)
    mwt = jnp.stack([mw_pad[kw:kw + Wout, :].T for kw in range(3)])
    w9t = w.reshape(9, C)
    body = functools.partial(_upconv_body, TH=TH, Hin=Hin, Win=Win, C=C)
    return pl.pallas_call(
        body,
        out_shape=jax.ShapeDtypeStruct((B, 1, Hout, Wout), f32),
        grid_spec=pltpu.PrefetchScalarGridSpec(
            num_scalar_prefetch=0,
            grid=(B, n_tiles),
            in_specs=[
                pl.BlockSpec((1, Hin, C, Win), lambda b, i: (b, 0, 0, 0)),
                pl.BlockSpec((Hout + 8, Hin), lambda b, i: (0, 0)),
                pl.BlockSpec((3, Win, Wout), lambda b, i: (0, 0, 0)),
                pl.BlockSpec((9, C), lambda b, i: (0, 0)),
                pl.BlockSpec((1, 1), lambda b, i: (0, 0)),
                pl.BlockSpec((1, 1), lambda b, i: (0, 0)),
            ],
            out_specs=pl.BlockSpec((1, 1, TH, Wout),
                                   lambda b, i: (b, 0, i, 0)),
            scratch_shapes=[pltpu.VMEM((TH + 2, 9, Win), f32)]),
        compiler_params=pltpu.CompilerParams(
            dimension_semantics=("parallel", "parallel")),
    )(x_hcw, mh_pad, mwt, w9t, scale.reshape(1, 1), shift.reshape(1, 1))


def kernel(x_nchw, text, t_w, t_b, e1_w, e1_scale, e1_shift, e2_wx, e2_wt,
           e2_scale, e2_shift, e3_wx, e3_wt, e3_scale, e3_shift, d1_w,
           d1_scale, d1_shift, d2_wx, d2_wt, d2_scale, d2_shift, d3a_wx,
           d3a_wt, d3a_scale, d3a_shift, d3b_w, d3b_scale, d3b_shift):
    B = x_nchw.shape[0]
    x = jnp.transpose(x_nchw, (0, 2, 3, 1)).astype(jnp.bfloat16)
    x = jnp.pad(x, ((0, 0), (0, 0), (0, 0), (0, 5)))
    text = text.astype(f32)

    layers = [(e2_wt, e2_scale, 0), (e3_wt, e3_scale, 64),
              (d2_wt, d2_scale, 128), (d3a_wt, d3a_scale, 192)]
    segs, bsegs = [], []
    for wt_, sc_, off in layers:
        ws = _wstar9(wt_) * sc_
        segs.append(jnp.einsum('kc,tco->kto', t_w[:, off:off + 64],
                               ws).reshape(512, -1))
        bsegs.append(jnp.einsum('c,tco->to', t_b[off:off + 64],
                                ws).reshape(-1))
    tball = _linear(text, jnp.concatenate(segs, axis=1),
                    jnp.concatenate(bsegs))
    tbs, o = [], 0
    for wt_, _, _ in layers:
        n = 9 * wt_.shape[-1]
        tbs.append(tball[:, o:o + n].reshape(B, 9, wt_.shape[-1]))
        o += n
    tb_e2, tb_e3, tb_d2, tb_d3a = tbs

    def r9(w):
        return w.reshape(9, w.shape[2], w.shape[3])

    e1w = jnp.pad(e1_w, ((0, 0), (0, 0), (0, 5), (0, 0)))
    h = _conv(x, r9(e1w), e1_scale, e1_shift, act='relu')
    h = _conv(h, r9(e2_wx), e2_scale, e2_shift, act='relu', tb=tb_e2)
    h = _conv(h, r9(e3_wx), e3_scale, e3_shift, act='relu', tb=tb_e3,
              pool=True)
    h = _conv(h, r9(d1_w), d1_scale, d1_shift, act='relu')
    h = _conv(h, r9(d2_wx), d2_scale, d2_shift, act='relu', tb=tb_d2)
    h = _conv(h, r9(d3a_wx), d3a_scale, d3a_shift, act='relu', tb=tb_d3a,
              hcw=True)
    return _upconv(h, d3b_w, d3b_scale, d3b_shift)


# speedup vs baseline: 1.7370x; 1.1196x over previous
import functools

import jax
import jax.numpy as jnp
from jax.experimental import pallas as pl
from jax.experimental.pallas import tpu as pltpu

f32 = jnp.float32


def _conv_body(*refs, TH, W, Wp, Cin, Cout, H, act, n_tiles, pool, has_tb,
               hcw=False, nchw_in=False):
    if has_tb:
        xm, xtop, xbot, w_ref, sc_ref, sh_ref, tb_ref, o_ref, xbuf = refs
    else:
        xm, xtop, xbot, w_ref, sc_ref, sh_ref, o_ref, xbuf = refs
        tb_ref = None
    i = pl.program_id(1)
    nf = (i > 0).astype(f32)
    nl = (i < n_tiles - 1).astype(f32)
    dt = xbuf.dtype
    xbuf[:, W:Wp, :] = jnp.zeros((TH + 4, Wp - W, Cin), dt)
    xbuf[0:1, 0:W, :] = jnp.zeros((1, W, Cin), dt)
    xbuf[TH + 3:TH + 4, 0:W, :] = jnp.zeros((1, W, Cin), dt)
    if nchw_in:
        C0 = xm.shape[1]
        xbuf[:, 0:W, C0:Cin] = jnp.zeros((TH + 4, W, Cin - C0), dt)
        xbuf[2:TH + 2, 0:W, 0:C0] = (
            jnp.transpose(xm[0], (1, 2, 0)).astype(dt))
        xbuf[1:2, 0:W, 0:C0] = (
            jnp.transpose(xtop[0][:, 7:8, :] * nf, (1, 2, 0)).astype(dt))
        xbuf[TH + 2:TH + 3, 0:W, 0:C0] = (
            jnp.transpose(xbot[0][:, 0:1, :] * nl, (1, 2, 0)).astype(dt))
    else:
        xbuf[2:TH + 2, 0:W, :] = xm[0]
        xbuf[1:2, 0:W, :] = (xtop[0] * nf).astype(dt)
        xbuf[TH + 2:TH + 3, 0:W, :] = (xbot[0] * nl).astype(dt)

    flat = xbuf[...].reshape((TH + 4) * Wp, Cin)
    acc = jnp.zeros((TH * Wp, Cout), f32)
    for dw in (-1, 0, 1):
        xs = flat[Wp + dw: Wp + dw + (TH + 2) * Wp, :]
        for kh in range(3):
            acc = acc + jnp.dot(xs[kh * Wp:(kh + TH) * Wp, :],
                                w_ref[kh * 3 + dw + 1],
                                preferred_element_type=f32)
    acc = acc * sc_ref[...] + sh_ref[...]
    a3 = acc.reshape(TH, Wp, Cout)

    if tb_ref is not None:
        tb = tb_ref[0]

        def v(k):
            return tb[k:k + 1, :].reshape(1, 1, Cout)

        rr = jax.lax.broadcasted_iota(jnp.int32, (TH, 1, 1), 0) + i * TH
        mt = (rr == 0).astype(f32)
        mb = (rr == H - 1).astype(f32)
        cc = jax.lax.broadcasted_iota(jnp.int32, (1, Wp, 1), 1)
        ml = (cc == 0).astype(f32)
        mr = (cc == W - 1).astype(f32)
        a3 = a3 + (v(0) + mt * v(1) + mb * v(2))
        a3 = a3 + ml * (v(3) + mt * v(5) + mb * v(7))
        a3 = a3 + mr * (v(4) + mt * v(6) + mb * v(8))

    if act == "relu":
        a3 = jnp.maximum(a3, 0.0)
    elif act == "sigmoid":
        a3 = 1.0 / (1.0 + jnp.exp(-a3))

    if pool:
        a = a3[:, 0:W, :].reshape(TH // 2, 2, W, Cout).max(axis=1)
        a = a.reshape(TH // 2, W // 2, 2, Cout).max(axis=2)
        o_ref[0] = a.astype(o_ref.dtype)
    elif hcw:
        o_ref[0] = jnp.swapaxes(a3[:, 0:W, :], 1, 2).astype(o_ref.dtype)
    else:
        o_ref[0] = a3[:, 0:W, :].astype(o_ref.dtype)


def _conv(x, w9, scale, shift, *, act, tb=None, pool=False, hcw=False,
          nchw_in=False, dt=None, TH=32):
    if nchw_in:
        B, C0, H, W = x.shape
        Cin = w9.shape[1]
    else:
        B, H, W, Cin = x.shape
    Cout = w9.shape[-1]
    TH = min(TH, H)
    n_tiles = H // TH
    Wp = W + 8
    dt = x.dtype if dt is None else dt
    body = functools.partial(
        _conv_body, TH=TH, W=W, Wp=Wp, Cin=Cin, Cout=Cout, H=H, act=act,
        n_tiles=n_tiles, pool=pool, has_tb=tb is not None, hcw=hcw,
        nchw_in=nchw_in)
    if nchw_in:
        nb = TH // 8
        in_specs = [
            pl.BlockSpec((1, C0, TH, W), lambda b, i: (b, 0, i, 0)),
            pl.BlockSpec((1, C0, 8, W),
                         lambda b, i: (b, 0, jnp.maximum(i * nb - 1, 0), 0)),
            pl.BlockSpec((1, C0, 8, W),
                         lambda b, i: (b, 0, jnp.minimum(i * nb + nb,
                                                         H // 8 - 1), 0)),
        ]
    else:
        in_specs = [
            pl.BlockSpec((1, TH, W, Cin), lambda b, i: (b, i, 0, 0)),
            pl.BlockSpec((1, 1, W, Cin),
                         lambda b, i: (b, jnp.maximum(i * TH - 1, 0), 0, 0)),
            pl.BlockSpec((1, 1, W, Cin),
                         lambda b, i: (b, jnp.minimum(i * TH + TH, H - 1),
                                       0, 0)),
        ]
    in_specs += [
        pl.BlockSpec((9, Cin, Cout), lambda b, i: (0, 0, 0)),
        pl.BlockSpec((1, Cout), lambda b, i: (0, 0)),
        pl.BlockSpec((1, Cout), lambda b, i: (0, 0)),
    ]
    inputs = [x, x, x, w9, scale.reshape(1, Cout), shift.reshape(1, Cout)]
    if tb is not None:
        in_specs.append(pl.BlockSpec((1, 9, Cout), lambda b, i: (b, 0, 0)))
        inputs.append(tb)
    out_dt = dt
    if pool:
        out_shape = jax.ShapeDtypeStruct((B, H // 2, W // 2, Cout), out_dt)
        out_spec = pl.BlockSpec((1, TH // 2, W // 2, Cout),
                                lambda b, i: (b, i, 0, 0))
    elif hcw:
        out_shape = jax.ShapeDtypeStruct((B, H, Cout, W), out_dt)
        out_spec = pl.BlockSpec((1, TH, Cout, W), lambda b, i: (b, i, 0, 0))
    else:
        out_shape = jax.ShapeDtypeStruct((B, H, W, Cout), out_dt)
        out_spec = pl.BlockSpec((1, TH, W, Cout), lambda b, i: (b, i, 0, 0))
    return pl.pallas_call(
        body,
        out_shape=out_shape,
        grid_spec=pltpu.PrefetchScalarGridSpec(
            num_scalar_prefetch=0,
            grid=(B, n_tiles),
            in_specs=in_specs,
            out_specs=out_spec,
            scratch_shapes=[pltpu.VMEM((TH + 4, Wp, Cin), dt)]),
        compiler_params=pltpu.CompilerParams(
            dimension_semantics=("parallel", "parallel")),
    )(*inputs)


def _lin_body(x_ref, w_ref, b_ref, o_ref):
    o_ref[...] = (jnp.dot(x_ref[...], w_ref[...],
                          preferred_element_type=f32) + b_ref[...])


def _linear(x, w, b):
    B, K = x.shape
    N = w.shape[-1]
    return pl.pallas_call(
        _lin_body,
        out_shape=jax.ShapeDtypeStruct((B, N), f32),
        grid=(1,),
        in_specs=[pl.BlockSpec((B, K), lambda i: (0, 0)),
                  pl.BlockSpec((K, N), lambda i: (0, 0)),
                  pl.BlockSpec((1, N), lambda i: (0, 0))],
        out_specs=pl.BlockSpec((B, N), lambda i: (0, 0)),
    )(x, w, b.reshape(1, N))


def _wstar9(wt):
    full = wt.sum((0, 1))
    top = -wt[0].sum(0)
    bot = -wt[2].sum(0)
    left = -wt[:, 0].sum(0)
    right = -wt[:, 2].sum(0)
    return jnp.stack([full, top, bot, left, right,
                      wt[0, 0], wt[0, 2], wt[2, 0], wt[2, 2]])


def _bilin_mat(n_in, n_out):
    i = jnp.arange(n_out, dtype=f32)
    src = i * (n_in - 1) / (n_out - 1)
    i0 = jnp.clip(jnp.floor(src).astype(jnp.int32), 0, n_in - 2)
    frac = src - i0.astype(f32)
    rows = jnp.arange(n_out)
    M = jnp.zeros((n_out, n_in), f32)
    M = M.at[rows, i0].add(1.0 - frac)
    M = M.at[rows, i0 + 1].add(frac)
    return M


def _upconv_body(x_ref, mh_ref, mwt_ref, w9t_ref, sc_ref, sh_ref, o_ref,
                 zbuf, *, TH, Hin, Win, C):
    i = pl.program_id(1)
    xflat = x_ref[0].reshape(Hin, C * Win)
    mh = mh_ref[pl.ds(i * TH, TH + 2), :]
    uph = jnp.dot(mh, xflat, preferred_element_type=f32)
    uph3 = uph.reshape(TH + 2, C, Win)
    w9t = w9t_ref[...]
    for r in range(TH + 2):
        zbuf[r] = jnp.dot(w9t, uph3[r], preferred_element_type=f32)
    zb = zbuf[...]
    acc = jnp.zeros((TH, 2 * Win), f32)
    for kh in range(3):
        for kw in range(3):
            zs = zb[kh:kh + TH, 3 * kh + kw, :]
            acc = acc + jnp.dot(zs, mwt_ref[kw],
                                preferred_element_type=f32)
    acc = acc * sc_ref[0, 0] + sh_ref[0, 0]
    o_ref[0, 0] = 1.0 / (1.0 + jnp.exp(-acc))


def _upconv(x_hcw, w, scale, shift, *, TH=32):
    B, Hin, C, Win = x_hcw.shape
    Hout, Wout = 2 * Hin, 2 * Win
    TH = min(TH, Hout)
    n_tiles = Hout // TH
    Mh = _bilin_mat(Hin, Hout)
    mh_pad = jnp.zeros((Hout + 8, Hin), f32).at[1:Hout + 1, :].set(Mh)
    Mw = _bilin_mat(Win, Wout)
    mw_pad = jnp.zeros((Wout + 2, Win), f32).at[1:Wout + 1, :].set(Mw)
    mwt = jnp.stack([mw_pad[kw:kw + Wout, :].T for kw in range(3)])
    w9t = w.reshape(9, C)
    body = functools.partial(_upconv_body, TH=TH, Hin=Hin, Win=Win, C=C)
    return pl.pallas_call(
        body,
        out_shape=jax.ShapeDtypeStruct((B, 1, Hout, Wout), f32),
        grid_spec=pltpu.PrefetchScalarGridSpec(
            num_scalar_prefetch=0,
            grid=(B, n_tiles),
            in_specs=[
                pl.BlockSpec((1, Hin, C, Win), lambda b, i: (b, 0, 0, 0)),
                pl.BlockSpec((Hout + 8, Hin), lambda b, i: (0, 0)),
                pl.BlockSpec((3, Win, Wout), lambda b, i: (0, 0, 0)),
                pl.BlockSpec((9, C), lambda b, i: (0, 0)),
                pl.BlockSpec((1, 1), lambda b, i: (0, 0)),
                pl.BlockSpec((1, 1), lambda b, i: (0, 0)),
            ],
            out_specs=pl.BlockSpec((1, 1, TH, Wout),
                                   lambda b, i: (b, 0, i, 0)),
            scratch_shapes=[pltpu.VMEM((TH + 2, 9, Win), f32)]),
        compiler_params=pltpu.CompilerParams(
            dimension_semantics=("parallel", "parallel")),
    )(x_hcw, mh_pad, mwt, w9t, scale.reshape(1, 1), shift.reshape(1, 1))


def kernel(x_nchw, text, t_w, t_b, e1_w, e1_scale, e1_shift, e2_wx, e2_wt,
           e2_scale, e2_shift, e3_wx, e3_wt, e3_scale, e3_shift, d1_w,
           d1_scale, d1_shift, d2_wx, d2_wt, d2_scale, d2_shift, d3a_wx,
           d3a_wt, d3a_scale, d3a_shift, d3b_w, d3b_scale, d3b_shift):
    B = x_nchw.shape[0]
    x = x_nchw.astype(f32)
    text = text.astype(f32)

    layers = [(e2_wt, e2_scale, 0), (e3_wt, e3_scale, 64),
              (d2_wt, d2_scale, 128), (d3a_wt, d3a_scale, 192)]
    segs, bsegs = [], []
    for wt_, sc_, off in layers:
        ws = _wstar9(wt_) * sc_
        segs.append(jnp.einsum('kc,tco->kto', t_w[:, off:off + 64],
                               ws).reshape(512, -1))
        bsegs.append(jnp.einsum('c,tco->to', t_b[off:off + 64],
                                ws).reshape(-1))
    tball = _linear(text, jnp.concatenate(segs, axis=1),
                    jnp.concatenate(bsegs))
    tbs, o = [], 0
    for wt_, _, _ in layers:
        n = 9 * wt_.shape[-1]
        tbs.append(tball[:, o:o + n].reshape(B, 9, wt_.shape[-1]))
        o += n
    tb_e2, tb_e3, tb_d2, tb_d3a = tbs

    def r9(w):
        return w.reshape(9, w.shape[2], w.shape[3])

    e1w = jnp.pad(e1_w, ((0, 0), (0, 0), (0, 5), (0, 0)))
    h = _conv(x, r9(e1w), e1_scale, e1_shift, act='relu', nchw_in=True,
              dt=jnp.bfloat16)
    h = _conv(h, r9(e2_wx), e2_scale, e2_shift, act='relu', tb=tb_e2)
    h = _conv(h, r9(e3_wx), e3_scale, e3_shift, act='relu', tb=tb_e3,
              pool=True)
    h = _conv(h, r9(d1_w), d1_scale, d1_shift, act='relu')
    h = _conv(h, r9(d2_wx), d2_scale, d2_shift, act='relu', tb=tb_d2)
    h = _conv(h, r9(d3a_wx), d3a_scale, d3a_shift, act='relu', tb=tb_d3a,
              hcw=True)
    return _upconv(h, d3b_w, d3b_scale, d3b_shift)


# speedup vs baseline: 1.7401x; 1.0018x over previous
import functools

import jax
import jax.numpy as jnp
from jax.experimental import pallas as pl
from jax.experimental.pallas import tpu as pltpu

f32 = jnp.float32


def _conv_body(*refs, TH, W, Wp, Cin, Cout, H, act, n_tiles, pool, has_tb,
               hcw=False, nchw_in=False):
    if has_tb:
        xm, xtop, xbot, w_ref, sc_ref, sh_ref, tb_ref, o_ref, xbuf = refs
    else:
        xm, xtop, xbot, w_ref, sc_ref, sh_ref, o_ref, xbuf = refs
        tb_ref = None
    i = pl.program_id(1)
    nf = (i > 0).astype(f32)
    nl = (i < n_tiles - 1).astype(f32)
    dt = xbuf.dtype
    xbuf[:, W:Wp, :] = jnp.zeros((TH + 4, Wp - W, Cin), dt)
    xbuf[0:1, 0:W, :] = jnp.zeros((1, W, Cin), dt)
    xbuf[TH + 3:TH + 4, 0:W, :] = jnp.zeros((1, W, Cin), dt)
    if nchw_in:
        C0 = xm.shape[1]
        xbuf[:, 0:W, C0:Cin] = jnp.zeros((TH + 4, W, Cin - C0), dt)
        xbuf[2:TH + 2, 0:W, 0:C0] = (
            jnp.transpose(xm[0], (1, 2, 0)).astype(dt))
        xbuf[1:2, 0:W, 0:C0] = (
            jnp.transpose(xtop[0][:, 7:8, :] * nf, (1, 2, 0)).astype(dt))
        xbuf[TH + 2:TH + 3, 0:W, 0:C0] = (
            jnp.transpose(xbot[0][:, 0:1, :] * nl, (1, 2, 0)).astype(dt))
    else:
        xbuf[2:TH + 2, 0:W, :] = xm[0]
        xbuf[1:2, 0:W, :] = (xtop[0] * nf).astype(dt)
        xbuf[TH + 2:TH + 3, 0:W, :] = (xbot[0] * nl).astype(dt)

    flat = xbuf[...].reshape((TH + 4) * Wp, Cin)
    acc = jnp.zeros((TH * Wp, Cout), f32)
    for dw in (-1, 0, 1):
        xs = flat[Wp + dw: Wp + dw + (TH + 2) * Wp, :]
        for kh in range(3):
            acc = acc + jnp.dot(xs[kh * Wp:(kh + TH) * Wp, :],
                                w_ref[kh * 3 + dw + 1],
                                preferred_element_type=f32)
    acc = acc * sc_ref[...] + sh_ref[...]
    a3 = acc.reshape(TH, Wp, Cout)

    if tb_ref is not None:
        tb = tb_ref[0]

        def v(k):
            return tb[k:k + 1, :].reshape(1, 1, Cout)

        rr = jax.lax.broadcasted_iota(jnp.int32, (TH, 1, 1), 0) + i * TH
        mt = (rr == 0).astype(f32)
        mb = (rr == H - 1).astype(f32)
        cc = jax.lax.broadcasted_iota(jnp.int32, (1, Wp, 1), 1)
        ml = (cc == 0).astype(f32)
        mr = (cc == W - 1).astype(f32)
        a3 = a3 + (v(0) + mt * v(1) + mb * v(2))
        a3 = a3 + ml * (v(3) + mt * v(5) + mb * v(7))
        a3 = a3 + mr * (v(4) + mt * v(6) + mb * v(8))

    if act == "relu":
        a3 = jnp.maximum(a3, 0.0)
    elif act == "sigmoid":
        a3 = 1.0 / (1.0 + jnp.exp(-a3))

    if pool:
        a = a3[:, 0:W, :].reshape(TH // 2, 2, W, Cout).max(axis=1)
        a = a.reshape(TH // 2, W // 2, 2, Cout).max(axis=2)
        o_ref[0] = a.astype(o_ref.dtype)
    elif hcw:
        o_ref[0] = jnp.swapaxes(a3[:, 0:W, :], 1, 2).astype(o_ref.dtype)
    else:
        o_ref[0] = a3[:, 0:W, :].astype(o_ref.dtype)


def _conv(x, w9, scale, shift, *, act, tb=None, pool=False, hcw=False,
          nchw_in=False, dt=None, TH=32):
    if nchw_in:
        B, C0, H, W = x.shape
        Cin = w9.shape[1]
    else:
        B, H, W, Cin = x.shape
    Cout = w9.shape[-1]
    TH = min(TH, H)
    n_tiles = H // TH
    Wp = W + 8
    dt = x.dtype if dt is None else dt
    body = functools.partial(
        _conv_body, TH=TH, W=W, Wp=Wp, Cin=Cin, Cout=Cout, H=H, act=act,
        n_tiles=n_tiles, pool=pool, has_tb=tb is not None, hcw=hcw,
        nchw_in=nchw_in)
    if nchw_in:
        nb = TH // 8
        in_specs = [
            pl.BlockSpec((1, C0, TH, W), lambda b, i: (b, 0, i, 0)),
            pl.BlockSpec((1, C0, 8, W),
                         lambda b, i: (b, 0, jnp.maximum(i * nb - 1, 0), 0)),
            pl.BlockSpec((1, C0, 8, W),
                         lambda b, i: (b, 0, jnp.minimum(i * nb + nb,
                                                         H // 8 - 1), 0)),
        ]
    else:
        in_specs = [
            pl.BlockSpec((1, TH, W, Cin), lambda b, i: (b, i, 0, 0)),
            pl.BlockSpec((1, 1, W, Cin),
                         lambda b, i: (b, jnp.maximum(i * TH - 1, 0), 0, 0)),
            pl.BlockSpec((1, 1, W, Cin),
                         lambda b, i: (b, jnp.minimum(i * TH + TH, H - 1),
                                       0, 0)),
        ]
    in_specs += [
        pl.BlockSpec((9, Cin, Cout), lambda b, i: (0, 0, 0)),
        pl.BlockSpec((1, Cout), lambda b, i: (0, 0)),
        pl.BlockSpec((1, Cout), lambda b, i: (0, 0)),
    ]
    inputs = [x, x, x, w9, scale.reshape(1, Cout), shift.reshape(1, Cout)]
    if tb is not None:
        in_specs.append(pl.BlockSpec((1, 9, Cout), lambda b, i: (b, 0, 0)))
        inputs.append(tb)
    out_dt = dt
    if pool:
        out_shape = jax.ShapeDtypeStruct((B, H // 2, W // 2, Cout), out_dt)
        out_spec = pl.BlockSpec((1, TH // 2, W // 2, Cout),
                                lambda b, i: (b, i, 0, 0))
    elif hcw:
        out_shape = jax.ShapeDtypeStruct((B, H, Cout, W), out_dt)
        out_spec = pl.BlockSpec((1, TH, Cout, W), lambda b, i: (b, i, 0, 0))
    else:
        out_shape = jax.ShapeDtypeStruct((B, H, W, Cout), out_dt)
        out_spec = pl.BlockSpec((1, TH, W, Cout), lambda b, i: (b, i, 0, 0))
    return pl.pallas_call(
        body,
        out_shape=out_shape,
        grid_spec=pltpu.PrefetchScalarGridSpec(
            num_scalar_prefetch=0,
            grid=(B, n_tiles),
            in_specs=in_specs,
            out_specs=out_spec,
            scratch_shapes=[pltpu.VMEM((TH + 4, Wp, Cin), dt)]),
        compiler_params=pltpu.CompilerParams(
            dimension_semantics=("parallel", "parallel")),
    )(*inputs)


def _lin_body(x_ref, w_ref, b_ref, o_ref):
    o_ref[...] = (jnp.dot(x_ref[...], w_ref[...],
                          preferred_element_type=f32) + b_ref[...])


def _linear(x, w, b):
    B, K = x.shape
    N = w.shape[-1]
    return pl.pallas_call(
        _lin_body,
        out_shape=jax.ShapeDtypeStruct((B, N), f32),
        grid=(1,),
        in_specs=[pl.BlockSpec((B, K), lambda i: (0, 0)),
                  pl.BlockSpec((K, N), lambda i: (0, 0)),
                  pl.BlockSpec((1, N), lambda i: (0, 0))],
        out_specs=pl.BlockSpec((B, N), lambda i: (0, 0)),
    )(x, w, b.reshape(1, N))


def _wstar9(wt):
    full = wt.sum((0, 1))
    top = -wt[0].sum(0)
    bot = -wt[2].sum(0)
    left = -wt[:, 0].sum(0)
    right = -wt[:, 2].sum(0)
    return jnp.stack([full, top, bot, left, right,
                      wt[0, 0], wt[0, 2], wt[2, 0], wt[2, 2]])


def _bilin_mat(n_in, n_out):
    i = jnp.arange(n_out, dtype=f32)
    src = i * (n_in - 1) / (n_out - 1)
    i0 = jnp.clip(jnp.floor(src).astype(jnp.int32), 0, n_in - 2)
    frac = src - i0.astype(f32)
    rows = jnp.arange(n_out)
    M = jnp.zeros((n_out, n_in), f32)
    M = M.at[rows, i0].add(1.0 - frac)
    M = M.at[rows, i0 + 1].add(frac)
    return M


def _upconv_body(x_ref, mh_ref, mwt_ref, w9t_ref, sc_ref, sh_ref, o_ref,
                 zbuf, *, TH, Hin, Win, C):
    i = pl.program_id(1)
    xflat = x_ref[0].reshape(Hin, C * Win)
    mh = mh_ref[pl.ds(i * TH, TH + 2), :]
    uph = jnp.dot(mh, xflat, preferred_element_type=f32)
    uph3 = uph.reshape(TH + 2, C, Win)
    w9t = w9t_ref[...]
    for r in range(TH + 2):
        zbuf[r] = jnp.dot(w9t, uph3[r], preferred_element_type=f32)
    zb = zbuf[...]
    acc = jnp.zeros((TH, 2 * Win), f32)
    for kh in range(3):
        for kw in range(3):
            zs = zb[kh:kh + TH, 3 * kh + kw, :]
            acc = acc + jnp.dot(zs, mwt_ref[kw],
                                preferred_element_type=f32)
    acc = acc * sc_ref[0, 0] + sh_ref[0, 0]
    o_ref[0, 0] = 1.0 / (1.0 + jnp.exp(-acc))


def _upconv(x_hcw, w, scale, shift, *, TH=32):
    B, Hin, C, Win = x_hcw.shape
    Hout, Wout = 2 * Hin, 2 * Win
    TH = min(TH, Hout)
    n_tiles = Hout // TH
    Mh = _bilin_mat(Hin, Hout)
    mh_pad = jnp.zeros((Hout + 8, Hin), f32).at[1:Hout + 1, :].set(Mh)
    Mw = _bilin_mat(Win, Wout)
    mw_pad = jnp.zeros((Wout + 2, Win), f32).at[1:Wout + 1, :].set(Mw)
    mwt = jnp.stack([mw_pad[kw:kw + Wout, :].T for kw in range(3)])
    w9t = w.reshape(9, C)
    body = functools.partial(_upconv_body, TH=TH, Hin=Hin, Win=Win, C=C)
    return pl.pallas_call(
        body,
        out_shape=jax.ShapeDtypeStruct((B, 1, Hout, Wout), f32),
        grid_spec=pltpu.PrefetchScalarGridSpec(
            num_scalar_prefetch=0,
            grid=(B, n_tiles),
            in_specs=[
                pl.BlockSpec((1, Hin, C, Win), lambda b, i: (b, 0, 0, 0)),
                pl.BlockSpec((Hout + 8, Hin), lambda b, i: (0, 0)),
                pl.BlockSpec((3, Win, Wout), lambda b, i: (0, 0, 0)),
                pl.BlockSpec((9, C), lambda b, i: (0, 0)),
                pl.BlockSpec((1, 1), lambda b, i: (0, 0)),
                pl.BlockSpec((1, 1), lambda b, i: (0, 0)),
            ],
            out_specs=pl.BlockSpec((1, 1, TH, Wout),
                                   lambda b, i: (b, 0, i, 0)),
            scratch_shapes=[pltpu.VMEM((TH + 2, 9, Win), f32)]),
        compiler_params=pltpu.CompilerParams(
            dimension_semantics=("parallel", "parallel")),
    )(x_hcw, mh_pad, mwt, w9t, scale.reshape(1, 1), shift.reshape(1, 1))


def kernel(x_nchw, text, t_w, t_b, e1_w, e1_scale, e1_shift, e2_wx, e2_wt,
           e2_scale, e2_shift, e3_wx, e3_wt, e3_scale, e3_shift, d1_w,
           d1_scale, d1_shift, d2_wx, d2_wt, d2_scale, d2_shift, d3a_wx,
           d3a_wt, d3a_scale, d3a_shift, d3b_w, d3b_scale, d3b_shift):
    B = x_nchw.shape[0]
    x = x_nchw.astype(f32)
    text = text.astype(f32)

    layers = [(e2_wt, e2_scale, 0), (e3_wt, e3_scale, 64),
              (d2_wt, d2_scale, 128), (d3a_wt, d3a_scale, 192)]
    segs, bsegs = [], []
    for wt_, sc_, off in layers:
        ws = _wstar9(wt_) * sc_
        segs.append(jnp.einsum('kc,tco->kto', t_w[:, off:off + 64],
                               ws).reshape(512, -1))
        bsegs.append(jnp.einsum('c,tco->to', t_b[off:off + 64],
                                ws).reshape(-1))
    tball = _linear(text, jnp.concatenate(segs, axis=1),
                    jnp.concatenate(bsegs))
    tbs, o = [], 0
    for wt_, _, _ in layers:
        n = 9 * wt_.shape[-1]
        tbs.append(tball[:, o:o + n].reshape(B, 9, wt_.shape[-1]))
        o += n
    tb_e2, tb_e3, tb_d2, tb_d3a = tbs

    def r9(w):
        return w.reshape(9, w.shape[2], w.shape[3])

    e1w = jnp.pad(e1_w, ((0, 0), (0, 0), (0, 5), (0, 0)))
    h = _conv(x, r9(e1w), e1_scale, e1_shift, act='relu', nchw_in=True,
              dt=jnp.bfloat16)
    h = _conv(h, r9(e2_wx), e2_scale, e2_shift, act='relu', tb=tb_e2)
    h = _conv(h, r9(e3_wx), e3_scale, e3_shift, act='relu', tb=tb_e3,
              pool=True)
    h = _conv(h, r9(d1_w), d1_scale, d1_shift, act='relu', TH=64)
    h = _conv(h, r9(d2_wx), d2_scale, d2_shift, act='relu', tb=tb_d2,
              TH=64)
    h = _conv(h, r9(d3a_wx), d3a_scale, d3a_shift, act='relu', tb=tb_d3a,
              hcw=True)
    return _upconv(h, d3b_w, d3b_scale, d3b_shift)
